# baseline (speedup 1.0000x reference)
import functools

import numpy as np
import jax
import jax.numpy as jnp
from jax import lax
from jax.experimental import pallas as pl
from jax.experimental.pallas import tpu as pltpu
from jax.experimental.pallas import tpu_sc as plsc

_FIELD_DIMS = np.full(26, 40000, dtype=np.int64)
_OFFSETS = np.concatenate([[0], np.cumsum(_FIELD_DIMS)[:-1]]).astype(np.int32)

_BATCH = 16384
_NFIELD = 26
_DIM = 16
_B = _BATCH * _NFIELD
_NC = 2
_NS = 16
_NW = _NC * _NS
_BPW = _B // _NW
_NCHUNK = 8
_C = _BPW // _NCHUNK
_LANES = 16

_OFFS_TILED = np.tile(_OFFSETS, _BPW // _NFIELD)


@functools.partial(
    pl.kernel,
    mesh=plsc.VectorSubcoreMesh(core_axis_name="c", subcore_axis_name="s"),
    compiler_params=pltpu.CompilerParams(use_tc_tiling_on_sc=False),
    out_type=jax.ShapeDtypeStruct((_B, _DIM), jnp.float32),
    scratch_types=[
        pltpu.VMEM((_BPW,), jnp.int32),
        pltpu.VMEM((_BPW,), jnp.int32),
        pltpu.VMEM((_C, _DIM), jnp.float32),
        pltpu.VMEM((_C, _DIM), jnp.float32),
        pltpu.SemaphoreType.DMA,
        pltpu.SemaphoreType.DMA,
        pltpu.SemaphoreType.DMA,
    ],
)
def _embed_gather(x_hbm, offs_hbm, table_hbm, out_hbm,
                  idx_v, offs_v, rows0, rows1, gsem, osem0, osem1):
    wid = lax.axis_index("s") * _NC + lax.axis_index("c")
    base = wid * _BPW
    pltpu.sync_copy(x_hbm.at[pl.ds(base, _BPW)], idx_v)
    pltpu.sync_copy(offs_hbm, offs_v)

    def add_offsets(c):
        def body(i, carry):
            s = c * _C + i * _LANES
            idx_v[pl.ds(s, _LANES)] = idx_v[pl.ds(s, _LANES)] + offs_v[pl.ds(s, _LANES)]
            return carry
        lax.fori_loop(0, _C // _LANES, body, 0)

    rows = (rows0, rows1)
    osems = (osem0, osem1)
    out_cp = [None, None]
    add_offsets(0)
    for c in range(_NCHUNK):
        b = c % 2
        if out_cp[b] is not None:
            out_cp[b].wait()
        g = pltpu.async_copy(
            table_hbm.at[idx_v.at[pl.ds(c * _C, _C)]], rows[b], gsem)
        if c + 1 < _NCHUNK:
            add_offsets(c + 1)
        g.wait()
        out_cp[b] = pltpu.async_copy(
            rows[b], out_hbm.at[pl.ds(base + c * _C, _C)], osems[b])
    for b in range(2):
        if out_cp[b] is not None:
            out_cp[b].wait()


def kernel(x, table):
    xf = x.reshape(_B)
    offs = jnp.asarray(_OFFS_TILED)
    out = _embed_gather(xf, offs, table)
    return out.reshape(_BATCH, _NFIELD, _DIM)


# speedup vs baseline: 3.4554x; 3.4554x over previous
import functools

import numpy as np
import jax
import jax.numpy as jnp
from jax import lax
from jax.experimental import pallas as pl
from jax.experimental.pallas import tpu as pltpu
from jax.experimental.pallas import tpu_sc as plsc

_BATCH = 16384
_NFIELD = 26
_DIM = 16
_ROWS = 1040000
_FIELD_SIZE = 40000
_NC = 2
_NS = 16
_NW = _NC * _NS
_BPW = _BATCH // _NW
_RT = _ROWS // 128
_LANES = 16
_NG = _BPW // _LANES

_EBASE = np.array([(e // 8) * _RT * 1024 + (e % 8) * 128 for e in range(_DIM)],
                  dtype=np.int32)


@functools.partial(
    pl.kernel,
    mesh=plsc.VectorSubcoreMesh(core_axis_name="c", subcore_axis_name="s"),
    compiler_params=pltpu.CompilerParams(use_tc_tiling_on_sc=False),
    out_type=jax.ShapeDtypeStruct((_NFIELD * 2 * 131072,), jnp.float32),
    scratch_types=[
        pltpu.VMEM((_BPW,), jnp.int32),
        pltpu.VMEM((_BPW,), jnp.int32),
        pltpu.VMEM((8192,), jnp.int32),
        pltpu.VMEM((8192,), jnp.int32),
        pltpu.VMEM((8192,), jnp.float32),
        pltpu.VMEM((8192,), jnp.float32),
        pltpu.SemaphoreType.DMA,
        pltpu.SemaphoreType.DMA,
        pltpu.SemaphoreType.DMA,
        pltpu.SemaphoreType.DMA,
    ],
)
def _embed_gather(xt_hbm, tbl_hbm, out_hbm,
                  xbuf, wbase, idx0, idx1, land0, land1,
                  gsem0, gsem1, osem0, osem1):
    wid = lax.axis_index("s") * _NC + lax.axis_index("c")
    b0 = wid * _BPW
    bt0 = wid * (_BPW // 128)

    def build_indices(f, idx_v):
        pltpu.sync_copy(xt_hbm.at[pl.ds(f * _BATCH + b0, _BPW)], xbuf)
        foff = f * _FIELD_SIZE

        def wb(g, carry):
            s = g * _LANES
            r = xbuf[pl.ds(s, _LANES)] + foff
            wbase[pl.ds(s, _LANES)] = ((r >> 7) << 10) + (r & 127)
            return carry
        lax.fori_loop(0, _NG, wb, 0)

        def grp(g, carry):
            base = wbase[pl.ds(g * _LANES, _LANES)]
            bt = g // 8
            blg = g % 8
            for eg in range(2):
                for es in range(8):
                    dst = (eg * 4 + bt) * 1024 + es * 128 + blg * _LANES
                    idx_v[pl.ds(dst, _LANES)] = base + int(_EBASE[eg * 8 + es])
            return carry
        lax.fori_loop(0, _NG, grp, 0)

    idxs = (idx0, idx1)
    lands = (land0, land1)
    gsems = (gsem0, gsem1)
    osems = (osem0, osem1)

    build_indices(0, idx0)
    g_prev = pltpu.async_copy(tbl_hbm.at[idx0], land0, gsem0)
    o_prev = [None, None]
    for f in range(_NFIELD):
        p = f % 2
        q = (f + 1) % 2
        if f + 1 < _NFIELD:
            if o_prev[q] is not None:
                o_prev[q][0].wait()
                o_prev[q][1].wait()
            build_indices(f + 1, idxs[q])
            g_next = pltpu.async_copy(tbl_hbm.at[idxs[q]], lands[q], gsems[q])
        g_prev.wait()
        o_prev[p] = (
            pltpu.async_copy(lands[p].at[pl.ds(0, 4096)],
                             out_hbm.at[pl.ds(f * 262144 + bt0 * 1024, 4096)],
                             osems[p]),
            pltpu.async_copy(lands[p].at[pl.ds(4096, 4096)],
                             out_hbm.at[pl.ds(f * 262144 + 131072 + bt0 * 1024,
                                              4096)],
                             osems[p]),
        )
        if f + 1 < _NFIELD:
            g_prev = g_next
    for p in range(2):
        if o_prev[p] is not None:
            o_prev[p][0].wait()
            o_prev[p][1].wait()


def kernel(x, table):
    tbl = table.T.reshape(2, 8, _RT, 128).transpose(0, 2, 1, 3).reshape(-1)
    xt = x.T.reshape(-1)
    out1 = _embed_gather(xt, tbl)
    out5 = out1.reshape(_NFIELD, 2, 128, 8, 128)
    return out5.transpose(2, 4, 0, 1, 3).reshape(_BATCH, _NFIELD, _DIM)
